# baseline (speedup 1.0000x reference)
import functools

import jax
import jax.numpy as jnp
from jax import lax
from jax.experimental import pallas as pl
from jax.experimental.pallas import tpu as pltpu
from jax.experimental.pallas import tpu_sc as plsc

NC = 2
NS = 16
L = 16
NW = NC * NS
B = 16384
BPW = B // NW
CH = 128
NCHUNK = BPW // CH
D = 64
W = 2 * D
NROW = 1_000_000 // 2
GBLK = 1_000_000 // D


def _rsqrt(x):
    xi = plsc.bitcast(x, jnp.int32)
    yi = jnp.int32(0x5F3759DF) - (xi >> 1)
    y = plsc.bitcast(yi, jnp.float32)
    xh = x * 0.5
    for _ in range(3):
        y = y * (1.5 - xh * y * y)
    return y


_mesh = plsc.VectorSubcoreMesh(core_axis_name="c", subcore_axis_name="s")

_CP = pltpu.CompilerParams(
    needs_layout_passes=False, use_tc_tiling_on_sc=False)


def _wu_body(u3, vu_hbm, gu_hbm, ws_hbm,
             idx_u, sup_u, blk_u, row_v, g_v_, ws_v, sem):
    wid = lax.axis_index("s") * NC + lax.axis_index("c")
    pltpu.sync_copy(u3.at[wid], idx_u)
    for k in range(NCHUNK):
        for c in range(CH // L):
            sl = pl.ds(c * L, L)
            iu = idx_u[k, sl]
            sup_u[k, sl] = iu >> 1
            blk_u[k, sl] = iu >> 6

    lanes = lax.iota(jnp.int32, L)
    zf = jnp.zeros((L,), jnp.float32)
    m1 = jnp.full((L,), 1, jnp.int32)
    m63 = jnp.full((L,), 63, jnp.int32)

    for k in range(NCHUNK):
        d1 = pltpu.async_copy(vu_hbm.at[sup_u.at[k]], row_v, sem)
        d2 = pltpu.async_copy(gu_hbm.at[blk_u.at[k]], g_v_, sem)
        d1.wait()
        d2.wait()

        def group_body(g, _, k=k):
            rows = g * L + lanes
            chunk = jnp.full((L,), k, jnp.int32)
            iu = plsc.load_gather(idx_u, [chunk, g * L + lanes])
            pu = (iu & m1) * D

            def d_ss(dd, ssu):
                wu = plsc.load_gather(row_v, [rows, pu + dd])
                return ssu + wu * wu

            ssu = lax.fori_loop(0, D, d_ss, zf)
            gu = plsc.load_gather(g_v_, [rows, iu & m63])
            su = gu * _rsqrt(ssu)

            def d_scale(dd, _):
                wu = plsc.load_gather(row_v, [rows, pu + dd])
                col = jnp.full((L,), 0, jnp.int32) + dd
                plsc.store_scatter(ws_v, [rows, col], wu * su)
                return 0

            lax.fori_loop(0, D, d_scale, 0)
            return 0

        lax.fori_loop(0, CH // L, group_body, 0)
        pltpu.sync_copy(ws_v, ws_hbm.at[pl.ds(wid * BPW + k * CH, CH)])


_wu_sc = functools.partial(
    pl.kernel,
    out_type=jax.ShapeDtypeStruct((B, D), jnp.float32),
    mesh=_mesh,
    compiler_params=_CP,
    scratch_types=[
        pltpu.VMEM((NCHUNK, CH), jnp.int32),
        pltpu.VMEM((NCHUNK, CH), jnp.int32),
        pltpu.VMEM((NCHUNK, CH), jnp.int32),
        pltpu.VMEM((CH, W), jnp.float32),
        pltpu.VMEM((CH, D), jnp.float32),
        pltpu.VMEM((CH, D), jnp.float32),
        pltpu.SemaphoreType.DMA,
    ],
)(_wu_body)


def _pred_body(i3, j3, vv_hbm, gv_hbm, ws_hbm, out_hbm,
               idx_i, idx_j, sup_i, sup_j, blk_i, blk_j,
               hi_v, hj_v, gi_v, gj_v, ws_v, obuf, sem):
    wid = lax.axis_index("s") * NC + lax.axis_index("c")
    pltpu.sync_copy(i3.at[wid], idx_i)
    pltpu.sync_copy(j3.at[wid], idx_j)
    for k in range(NCHUNK):
        for c in range(CH // L):
            sl = pl.ds(c * L, L)
            ii = idx_i[k, sl]
            ij = idx_j[k, sl]
            sup_i[k, sl] = ii >> 1
            sup_j[k, sl] = ij >> 1
            blk_i[k, sl] = ii >> 6
            blk_j[k, sl] = ij >> 6

    lanes = lax.iota(jnp.int32, L)
    zf = jnp.zeros((L,), jnp.float32)
    m1 = jnp.full((L,), 1, jnp.int32)
    m63 = jnp.full((L,), 63, jnp.int32)

    pi_acc = zf
    pj_acc = zf
    for k in range(NCHUNK):
        descs = [
            pltpu.async_copy(vv_hbm.at[sup_i.at[k]], hi_v, sem),
            pltpu.async_copy(vv_hbm.at[sup_j.at[k]], hj_v, sem),
            pltpu.async_copy(gv_hbm.at[blk_i.at[k]], gi_v, sem),
            pltpu.async_copy(gv_hbm.at[blk_j.at[k]], gj_v, sem),
        ]
        pltpu.sync_copy(ws_hbm.at[pl.ds(wid * BPW + k * CH, CH)], ws_v)
        for dsc in descs:
            dsc.wait()

        def group_body(g, accs, k=k):
            pi_a, pj_a = accs
            rows = g * L + lanes
            chunk = jnp.full((L,), k, jnp.int32)
            icol = g * L + lanes
            ii = plsc.load_gather(idx_i, [chunk, icol])
            ij = plsc.load_gather(idx_j, [chunk, icol])
            pi_ = (ii & m1) * D
            pj_ = (ij & m1) * D

            def d_body(dd, carry):
                ssi, ssj, qi, qj = carry
                col = jnp.full((L,), 0, jnp.int32) + dd
                ws = plsc.load_gather(ws_v, [rows, col])
                hi = plsc.load_gather(hi_v, [rows, pi_ + dd])
                hj = plsc.load_gather(hj_v, [rows, pj_ + dd])
                return (ssi + hi * hi, ssj + hj * hj,
                        qi + ws * hi, qj + ws * hj)

            ssi, ssj, qi, qj = lax.fori_loop(0, D, d_body, (zf, zf, zf, zf))
            gi = plsc.load_gather(gi_v, [rows, ii & m63])
            gj = plsc.load_gather(gj_v, [rows, ij & m63])
            pi_a = pi_a + (gi * _rsqrt(ssi)) * qi
            pj_a = pj_a + (gj * _rsqrt(ssj)) * qj
            return (pi_a, pj_a)

        pi_acc, pj_acc = lax.fori_loop(
            0, CH // L, group_body, (pi_acc, pj_acc))

    pi_s = jnp.sum(pi_acc)
    pj_s = jnp.sum(pj_acc)
    obuf[...] = jnp.where(lanes == 0, pi_s,
                          jnp.where(lanes == 1, pj_s, 0.0))
    pltpu.sync_copy(obuf, out_hbm.at[wid])


_pred_sc = functools.partial(
    pl.kernel,
    out_type=jax.ShapeDtypeStruct((NW, L), jnp.float32),
    mesh=_mesh,
    compiler_params=_CP,
    scratch_types=[
        pltpu.VMEM((NCHUNK, CH), jnp.int32),
        pltpu.VMEM((NCHUNK, CH), jnp.int32),
        pltpu.VMEM((NCHUNK, CH), jnp.int32),
        pltpu.VMEM((NCHUNK, CH), jnp.int32),
        pltpu.VMEM((NCHUNK, CH), jnp.int32),
        pltpu.VMEM((NCHUNK, CH), jnp.int32),
        pltpu.VMEM((CH, W), jnp.float32),
        pltpu.VMEM((CH, W), jnp.float32),
        pltpu.VMEM((CH, D), jnp.float32),
        pltpu.VMEM((CH, D), jnp.float32),
        pltpu.VMEM((CH, D), jnp.float32),
        pltpu.VMEM((L,), jnp.float32),
        pltpu.SemaphoreType.DMA,
    ],
)(_pred_body)


def kernel(u, i, j, v_u, g_u, v_v, g_v):
    u3 = u.astype(jnp.int32).reshape(NW, NCHUNK, CH)
    i3 = i.astype(jnp.int32).reshape(NW, NCHUNK, CH)
    j3 = j.astype(jnp.int32).reshape(NW, NCHUNK, CH)
    vu2 = jnp.reshape(v_u, (NROW, W))
    vv2 = jnp.reshape(v_v, (NROW, W))
    gu2 = jnp.reshape(g_u, (GBLK, D))
    gv2 = jnp.reshape(g_v, (GBLK, D))
    ws = _wu_sc(u3, vu2, gu2)
    partials = _pred_sc(i3, j3, vv2, gv2, ws)
    return (jnp.sum(partials[:, 0]), jnp.sum(partials[:, 1]))


# speedup vs baseline: 1.0260x; 1.0260x over previous
import functools

import jax
import jax.numpy as jnp
from jax import lax
from jax.experimental import pallas as pl
from jax.experimental.pallas import tpu as pltpu
from jax.experimental.pallas import tpu_sc as plsc

NC = 2
NS = 16
L = 16
NW = NC * NS
B = 16384
BPW = B // NW
CH = 128
NCHUNK = BPW // CH
D = 64
W = 2 * D
NROW = 1_000_000 // 2
GBLK = 1_000_000 // D


def _rsqrt(x):
    xi = plsc.bitcast(x, jnp.int32)
    yi = jnp.int32(0x5F3759DF) - (xi >> 1)
    y = plsc.bitcast(yi, jnp.float32)
    xh = x * 0.5
    for _ in range(3):
        y = y * (1.5 - xh * y * y)
    return y


_mesh = plsc.VectorSubcoreMesh(core_axis_name="c", subcore_axis_name="s")

_CP = pltpu.CompilerParams(
    needs_layout_passes=False, use_tc_tiling_on_sc=False)


def _bpr_body(u3, i3, j3, vu_hbm, gu_hbm, vv_hbm, gv_hbm, out_hbm,
              idx_u, idx_i, idx_j, sup_u, sup_i, sup_j,
              blk_u, blk_i, blk_j,
              hu_v, hi_v, hj_v, gu_v, gi_v, gj_v, obuf, sem):
    wid = lax.axis_index("s") * NC + lax.axis_index("c")
    pltpu.sync_copy(u3.at[wid], idx_u)
    pltpu.sync_copy(i3.at[wid], idx_i)
    pltpu.sync_copy(j3.at[wid], idx_j)
    for k in range(NCHUNK):
        for c in range(CH // L):
            sl = pl.ds(c * L, L)
            iu = idx_u[k, sl]
            ii = idx_i[k, sl]
            ij = idx_j[k, sl]
            sup_u[k, sl] = iu >> 1
            sup_i[k, sl] = ii >> 1
            sup_j[k, sl] = ij >> 1
            blk_u[k, sl] = iu >> 6
            blk_i[k, sl] = ii >> 6
            blk_j[k, sl] = ij >> 6

    lanes = lax.iota(jnp.int32, L)
    zf = jnp.zeros((L,), jnp.float32)
    m1 = jnp.full((L,), 1, jnp.int32)
    m63 = jnp.full((L,), 63, jnp.int32)

    pi_acc = zf
    pj_acc = zf
    for k in range(NCHUNK):
        descs = [
            pltpu.async_copy(vu_hbm.at[sup_u.at[k]], hu_v, sem),
            pltpu.async_copy(vv_hbm.at[sup_i.at[k]], hi_v, sem),
            pltpu.async_copy(vv_hbm.at[sup_j.at[k]], hj_v, sem),
            pltpu.async_copy(gu_hbm.at[blk_u.at[k]], gu_v, sem),
            pltpu.async_copy(gv_hbm.at[blk_i.at[k]], gi_v, sem),
            pltpu.async_copy(gv_hbm.at[blk_j.at[k]], gj_v, sem),
        ]
        for dsc in descs:
            dsc.wait()

        def group_body(g, accs, k=k):
            pi_a, pj_a = accs
            rows = g * L + lanes
            chunk = jnp.full((L,), k, jnp.int32)
            icol = g * L + lanes
            iu = plsc.load_gather(idx_u, [chunk, icol])
            ii = plsc.load_gather(idx_i, [chunk, icol])
            ij = plsc.load_gather(idx_j, [chunk, icol])
            pu_ = (iu & m1) * D
            pi_ = (ii & m1) * D
            pj_ = (ij & m1) * D

            def d_body(dd, carry):
                ssu, ssi, ssj, qi, qj = carry
                wu = plsc.load_gather(hu_v, [rows, pu_ + dd])
                hi = plsc.load_gather(hi_v, [rows, pi_ + dd])
                hj = plsc.load_gather(hj_v, [rows, pj_ + dd])
                return (ssu + wu * wu, ssi + hi * hi, ssj + hj * hj,
                        qi + wu * hi, qj + wu * hj)

            ssu, ssi, ssj, qi, qj = lax.fori_loop(
                0, D, d_body, (zf, zf, zf, zf, zf))
            gu = plsc.load_gather(gu_v, [rows, iu & m63])
            gi = plsc.load_gather(gi_v, [rows, ii & m63])
            gj = plsc.load_gather(gj_v, [rows, ij & m63])
            su = gu * _rsqrt(ssu)
            pi_a = pi_a + su * (gi * _rsqrt(ssi)) * qi
            pj_a = pj_a + su * (gj * _rsqrt(ssj)) * qj
            return (pi_a, pj_a)

        pi_acc, pj_acc = lax.fori_loop(
            0, CH // L, group_body, (pi_acc, pj_acc))

    pi_s = jnp.sum(pi_acc)
    pj_s = jnp.sum(pj_acc)
    obuf[...] = jnp.where(lanes == 0, pi_s,
                          jnp.where(lanes == 1, pj_s, 0.0))
    pltpu.sync_copy(obuf, out_hbm.at[wid])


_bpr_sc = functools.partial(
    pl.kernel,
    out_type=jax.ShapeDtypeStruct((NW, L), jnp.float32),
    mesh=_mesh,
    compiler_params=_CP,
    scratch_types=[
        pltpu.VMEM((NCHUNK, CH), jnp.int32),
        pltpu.VMEM((NCHUNK, CH), jnp.int32),
        pltpu.VMEM((NCHUNK, CH), jnp.int32),
        pltpu.VMEM((NCHUNK, CH), jnp.int32),
        pltpu.VMEM((NCHUNK, CH), jnp.int32),
        pltpu.VMEM((NCHUNK, CH), jnp.int32),
        pltpu.VMEM((NCHUNK, CH), jnp.int32),
        pltpu.VMEM((NCHUNK, CH), jnp.int32),
        pltpu.VMEM((NCHUNK, CH), jnp.int32),
        pltpu.VMEM((CH, W), jnp.float32),
        pltpu.VMEM((CH, W), jnp.float32),
        pltpu.VMEM((CH, W), jnp.float32),
        pltpu.VMEM((CH, D), jnp.float32),
        pltpu.VMEM((CH, D), jnp.float32),
        pltpu.VMEM((CH, D), jnp.float32),
        pltpu.VMEM((L,), jnp.float32),
        pltpu.SemaphoreType.DMA,
    ],
)(_bpr_body)


def kernel(u, i, j, v_u, g_u, v_v, g_v):
    u3 = u.astype(jnp.int32).reshape(NW, NCHUNK, CH)
    i3 = i.astype(jnp.int32).reshape(NW, NCHUNK, CH)
    j3 = j.astype(jnp.int32).reshape(NW, NCHUNK, CH)
    vu2 = jnp.reshape(v_u, (NROW, W))
    vv2 = jnp.reshape(v_v, (NROW, W))
    gu2 = jnp.reshape(g_u, (GBLK, D))
    gv2 = jnp.reshape(g_v, (GBLK, D))
    partials = _bpr_sc(u3, i3, j3, vu2, gu2, vv2, gv2)
    return (jnp.sum(partials[:, 0]), jnp.sum(partials[:, 1]))


# speedup vs baseline: 1.0397x; 1.0134x over previous
import functools

import jax
import jax.numpy as jnp
from jax import lax
from jax.experimental import pallas as pl
from jax.experimental.pallas import tpu as pltpu
from jax.experimental.pallas import tpu_sc as plsc

NC = 2
NS = 16
L = 16
NW = NC * NS
B = 16384
BPW = B // NW
CH = 128
NCHUNK = BPW // CH
D = 64
W = 2 * D
NROW = 1_000_000 // 2
GBLK = 1_000_000 // D


def _rsqrt(x):
    xi = plsc.bitcast(x, jnp.int32)
    yi = jnp.int32(0x5F3759DF) - (xi >> 1)
    y = plsc.bitcast(yi, jnp.float32)
    xh = x * 0.5
    for _ in range(3):
        y = y * (1.5 - xh * y * y)
    return y


_mesh = plsc.VectorSubcoreMesh(core_axis_name="c", subcore_axis_name="s")

_CP = pltpu.CompilerParams(
    needs_layout_passes=False, use_tc_tiling_on_sc=False)


def _bpr_body(u3, i3, j3, vu_hbm, gu_hbm, vv_hbm, gv_hbm, out_hbm,
              idx_u, idx_i, idx_j, sup_u, sup_i, sup_j,
              blk_u, blk_i, blk_j,
              hu_v, hi_v, hj_v, gu_v, gi_v, gj_v, obuf, sem):
    wid = lax.axis_index("s") * NC + lax.axis_index("c")
    pltpu.sync_copy(u3.at[wid], idx_u)
    pltpu.sync_copy(i3.at[wid], idx_i)
    pltpu.sync_copy(j3.at[wid], idx_j)
    for k in range(NCHUNK):
        for c in range(CH // L):
            sl = pl.ds(c * L, L)
            iu = idx_u[k, sl]
            ii = idx_i[k, sl]
            ij = idx_j[k, sl]
            sup_u[k, sl] = iu
            sup_i[k, sl] = ii
            sup_j[k, sl] = ij
            blk_u[k, sl] = iu >> 6
            blk_i[k, sl] = ii >> 6
            blk_j[k, sl] = ij >> 6

    lanes = lax.iota(jnp.int32, L)
    zf = jnp.zeros((L,), jnp.float32)
    m1 = jnp.full((L,), 1, jnp.int32)
    m63 = jnp.full((L,), 63, jnp.int32)

    pi_acc = zf
    pj_acc = zf
    for k in range(NCHUNK):
        descs = [
            pltpu.async_copy(vu_hbm.at[sup_u.at[k]], hu_v, sem),
            pltpu.async_copy(vv_hbm.at[sup_i.at[k]], hi_v, sem),
            pltpu.async_copy(vv_hbm.at[sup_j.at[k]], hj_v, sem),
            pltpu.async_copy(gu_hbm.at[blk_u.at[k]], gu_v, sem),
            pltpu.async_copy(gv_hbm.at[blk_i.at[k]], gi_v, sem),
            pltpu.async_copy(gv_hbm.at[blk_j.at[k]], gj_v, sem),
        ]
        for dsc in descs:
            dsc.wait()

        def group_body(g, accs, k=k):
            pi_a, pj_a = accs
            rows = g * L + lanes
            chunk = jnp.full((L,), k, jnp.int32)
            icol = g * L + lanes
            iu = plsc.load_gather(idx_u, [chunk, icol])
            ii = plsc.load_gather(idx_i, [chunk, icol])
            ij = plsc.load_gather(idx_j, [chunk, icol])
            zc = jnp.zeros((L,), jnp.int32)
            pu_ = zc
            pi_ = zc
            pj_ = zc

            def d_body(dd, carry):
                ssu, ssi, ssj, qi, qj = carry
                wu = plsc.load_gather(hu_v, [rows, pu_ + dd])
                hi = plsc.load_gather(hi_v, [rows, pi_ + dd])
                hj = plsc.load_gather(hj_v, [rows, pj_ + dd])
                return (ssu + wu * wu, ssi + hi * hi, ssj + hj * hj,
                        qi + wu * hi, qj + wu * hj)

            ssu, ssi, ssj, qi, qj = lax.fori_loop(
                0, D, d_body, (zf, zf, zf, zf, zf))
            gu = plsc.load_gather(gu_v, [rows, iu & m63])
            gi = plsc.load_gather(gi_v, [rows, ii & m63])
            gj = plsc.load_gather(gj_v, [rows, ij & m63])
            su = gu * _rsqrt(ssu)
            pi_a = pi_a + su * (gi * _rsqrt(ssi)) * qi
            pj_a = pj_a + su * (gj * _rsqrt(ssj)) * qj
            return (pi_a, pj_a)

        pi_acc, pj_acc = lax.fori_loop(
            0, CH // L, group_body, (pi_acc, pj_acc))

    pi_s = jnp.sum(pi_acc)
    pj_s = jnp.sum(pj_acc)
    obuf[...] = jnp.where(lanes == 0, pi_s,
                          jnp.where(lanes == 1, pj_s, 0.0))
    pltpu.sync_copy(obuf, out_hbm.at[wid])


_bpr_sc = functools.partial(
    pl.kernel,
    out_type=jax.ShapeDtypeStruct((NW, L), jnp.float32),
    mesh=_mesh,
    compiler_params=_CP,
    scratch_types=[
        pltpu.VMEM((NCHUNK, CH), jnp.int32),
        pltpu.VMEM((NCHUNK, CH), jnp.int32),
        pltpu.VMEM((NCHUNK, CH), jnp.int32),
        pltpu.VMEM((NCHUNK, CH), jnp.int32),
        pltpu.VMEM((NCHUNK, CH), jnp.int32),
        pltpu.VMEM((NCHUNK, CH), jnp.int32),
        pltpu.VMEM((NCHUNK, CH), jnp.int32),
        pltpu.VMEM((NCHUNK, CH), jnp.int32),
        pltpu.VMEM((NCHUNK, CH), jnp.int32),
        pltpu.VMEM((CH, D), jnp.float32),
        pltpu.VMEM((CH, D), jnp.float32),
        pltpu.VMEM((CH, D), jnp.float32),
        pltpu.VMEM((CH, D), jnp.float32),
        pltpu.VMEM((CH, D), jnp.float32),
        pltpu.VMEM((CH, D), jnp.float32),
        pltpu.VMEM((L,), jnp.float32),
        pltpu.SemaphoreType.DMA,
    ],
)(_bpr_body)


def kernel(u, i, j, v_u, g_u, v_v, g_v):
    u3 = u.astype(jnp.int32).reshape(NW, NCHUNK, CH)
    i3 = i.astype(jnp.int32).reshape(NW, NCHUNK, CH)
    j3 = j.astype(jnp.int32).reshape(NW, NCHUNK, CH)
    vu2 = v_u
    vv2 = v_v
    gu2 = jnp.reshape(g_u, (GBLK, D))
    gv2 = jnp.reshape(g_v, (GBLK, D))
    partials = _bpr_sc(u3, i3, j3, vu2, gu2, vv2, gv2)
    return (jnp.sum(partials[:, 0]), jnp.sum(partials[:, 1]))
